# baseline (speedup 1.0000x reference)
import functools

import jax
import jax.numpy as jnp
from jax import lax
from jax.experimental import pallas as pl
from jax.experimental.pallas import tpu as pltpu
from jax.experimental.pallas import tpu_sc as plsc

BATCH, SEQ, VOCAB, DIM = 4096, 200, 1000000, 32
NC, NS = 2, 16
NW = NC * NS
RPW = BATCH // NW
TPW = RPW * SEQ
LANES = 16
SROWS = 512
NSTR = TPW // SROWS
NBUF = 5
SLOTS = RPW + 8
CDIM = 16
RDIV_MUL, RDIV_SHIFT = 41944, 23


def _body(ids_hbm, mask_hbm, table_hbm, out_hbm, ids_sv, mask_sv, rows_v,
          sidx_v, ones_v, res_v, cnt_v, acc_sh, cacc_sh, *sems):
    lsem = sems[:NBUF]
    gsem = sems[NBUF:2 * NBUF]
    ssem = sems[2 * NBUF:]
    sid = lax.axis_index("s")
    wid = sid * NC + lax.axis_index("c")
    slot0 = sid * SLOTS
    slot0_v = jnp.full((LANES,), slot0, jnp.int32)
    trash_v = jnp.full((LANES,), slot0 + RPW, jnp.int32)
    iota = lax.iota(jnp.int32, LANES)
    ones = jnp.full((LANES,), 1.0, jnp.float32)

    def zrow(r, _):
        res_v[r, pl.ds(0, LANES)] = jnp.zeros((LANES,), jnp.float32)
        res_v[r, pl.ds(LANES, LANES)] = jnp.zeros((LANES,), jnp.float32)
        cnt_v[r, pl.ds(0, LANES)] = jnp.zeros((LANES,), jnp.float32)
        return 0

    lax.fori_loop(0, RPW, zrow, 0)

    def orow(r, _):
        ones_v[r, pl.ds(0, LANES)] = ones
        return 0

    lax.fori_loop(0, SROWS, orow, 0)
    pltpu.sync_copy(res_v, acc_sh.at[pl.ds(slot0, RPW)])
    pltpu.sync_copy(res_v.at[pl.ds(0, SLOTS - RPW)],
                    acc_sh.at[pl.ds(slot0 + RPW, SLOTS - RPW)])
    pltpu.sync_copy(cnt_v, cacc_sh.at[pl.ds(slot0, RPW)])
    pltpu.sync_copy(cnt_v.at[pl.ds(0, SLOTS - RPW)],
                    cacc_sh.at[pl.ds(slot0 + RPW, SLOTS - RPW)])

    def fire_load(i, b):
        pltpu.async_copy(ids_hbm.at[wid, i], ids_sv.at[b], lsem[b])
        pltpu.async_copy(mask_hbm.at[wid, i], mask_sv.at[b], lsem[b])

    def wait_load(i, b):
        pltpu.make_async_copy(ids_hbm.at[wid, i], ids_sv.at[b],
                              lsem[b]).wait()
        pltpu.make_async_copy(mask_hbm.at[wid, i], mask_sv.at[b],
                              lsem[b]).wait()

    def fire_gather(b):
        pltpu.async_copy(table_hbm.at[ids_sv.at[b]], rows_v.at[b], gsem[b])

    def wait_gather(b):
        pltpu.make_async_copy(table_hbm.at[ids_sv.at[b]], rows_v.at[b],
                              gsem[b]).wait()

    def fire_scatter(b):
        pltpu.async_copy(rows_v.at[b], acc_sh.at[sidx_v.at[b]], ssem[b],
                         add=True)
        pltpu.async_copy(ones_v, cacc_sh.at[sidx_v.at[b]], ssem[b],
                         add=True)

    def wait_scatter(b):
        pltpu.make_async_copy(rows_v.at[b], acc_sh.at[sidx_v.at[b]],
                              ssem[b]).wait()
        pltpu.make_async_copy(ones_v, cacc_sh.at[sidx_v.at[b]],
                              ssem[b]).wait()

    def build_sidx(i, b):
        g00 = i * SROWS

        def chunk(cc, _):
            co = cc * LANES
            m = mask_sv[b, pl.ds(co, LANES)]
            gv = jnp.full((LANES,), g00 + co) + iota
            rowv = (gv * RDIV_MUL) >> RDIV_SHIFT
            sidx_v[b, pl.ds(co, LANES)] = jnp.where(
                m > 0, slot0_v + rowv, trash_v)
            return 0

        lax.fori_loop(0, SROWS // LANES, chunk, 0)

    for b in range(3):
        fire_load(b, b)
    for b in range(2):
        wait_load(b, b)
        fire_gather(b)

    def ring(q, _):
        for b in range(NBUF):
            i = q * NBUF + b

            @pl.when(i >= 2)
            def _():
                wait_scatter((b + 3) % NBUF)

            @pl.when(i + 3 < NSTR)
            def _():
                fire_load(i + 3, (b + 3) % NBUF)

            @pl.when(i + 2 < NSTR)
            def _():
                wait_load(i + 2, (b + 2) % NBUF)
                fire_gather((b + 2) % NBUF)

            wait_gather(b)
            build_sidx(i, b)
            fire_scatter(b)
        return 0

    lax.fori_loop(0, NSTR // NBUF, ring, 0)
    for b in range(NBUF - 2, NBUF):
        wait_scatter(b)

    pltpu.sync_copy(acc_sh.at[pl.ds(slot0, RPW)], res_v)
    pltpu.sync_copy(cacc_sh.at[pl.ds(slot0, RPW)], cnt_v)

    def div_row(j, _):
        inv = 1.0 / cnt_v[j, pl.ds(0, LANES)]
        res_v[j, pl.ds(0, LANES)] = res_v[j, pl.ds(0, LANES)] * inv
        res_v[j, pl.ds(LANES, LANES)] = (
            res_v[j, pl.ds(LANES, LANES)] * inv)
        return 0

    lax.fori_loop(0, RPW, div_row, 0)
    pltpu.sync_copy(res_v, out_hbm.at[pl.ds(wid * RPW, RPW)])


@functools.partial(
    pl.kernel,
    out_type=jax.ShapeDtypeStruct((BATCH, DIM), jnp.float32),
    mesh=plsc.VectorSubcoreMesh(core_axis_name="c", subcore_axis_name="s",
                                num_cores=NC, num_subcores=NS),
    compiler_params=pltpu.CompilerParams(use_tc_tiling_on_sc=False),
    scratch_types=[
        pltpu.VMEM((NBUF, SROWS), jnp.int32),
        pltpu.VMEM((NBUF, SROWS), jnp.int32),
        pltpu.VMEM((NBUF, SROWS, DIM), jnp.float32),
        pltpu.VMEM((NBUF, SROWS), jnp.int32),
        pltpu.VMEM((SROWS, CDIM), jnp.float32),
        pltpu.VMEM((RPW, DIM), jnp.float32),
        pltpu.VMEM((RPW, CDIM), jnp.float32),
        pltpu.VMEM_SHARED((NS * SLOTS, DIM), jnp.float32),
        pltpu.VMEM_SHARED((NS * SLOTS, CDIM), jnp.float32),
    ] + [pltpu.SemaphoreType.DMA] * (3 * NBUF),
)
def _encode(ids_hbm, mask_hbm, table_hbm, out_hbm, *refs):
    _body(ids_hbm, mask_hbm, table_hbm, out_hbm, *refs)


def kernel(input_ids, attention_mask, embedding_table):
    ids = input_ids.reshape(NW, NSTR, SROWS)
    mask = attention_mask.reshape(NW, NSTR, SROWS)
    return _encode(ids, mask, embedding_table)
